# baseline (speedup 1.0000x reference)
import functools

import jax
import jax.numpy as jnp
from jax import lax
from jax.experimental import pallas as pl
from jax.experimental.pallas import tpu as pltpu
from jax.experimental.pallas import tpu_sc as plsc

_NC = 2
_NS = 16
_NW = _NC * _NS
_C = 50
_NBUF = 3
_ZB = 128
_NPAD = 10240


@functools.lru_cache(maxsize=None)
def _make_segsum(per_w):
    rpt = _NPAD // _NS
    nz = rpt // _ZB

    scratch = [
        pltpu.VMEM_SHARED((_NPAD, 128), jnp.float32),
        pltpu.VMEM((per_w, _C), jnp.int32),
        pltpu.VMEM((per_w, _C), jnp.int32),
    ] + [pltpu.VMEM((_C, 128), jnp.float32) for _ in range(_NBUF)] + [
        pltpu.SemaphoreType.DMA for _ in range(_NBUF)
    ] + [
        pltpu.SemaphoreType.DMA,
        pltpu.SemaphoreType.DMA,
    ]

    mesh = plsc.VectorSubcoreMesh(core_axis_name="c", subcore_axis_name="s")

    ngrp = per_w // _NBUF
    rem = per_w - ngrp * _NBUF

    def body(h_hbm, src_hbm, dst_hbm, acc_out, acc_sh, sidx, didx, *rest):
        bufs = rest[:_NBUF]
        sems = rest[_NBUF:2 * _NBUF]
        sem_a, sem_b = rest[2 * _NBUF:]
        cid = lax.axis_index("c")
        sid = lax.axis_index("s")
        wid = cid * _NS + sid

        idx_a = pltpu.async_copy(src_hbm.at[wid], sidx, sem_a)
        idx_b = pltpu.async_copy(dst_hbm.at[wid], didx, sem_b)

        def _zb(i, carry):
            bufs[0][i // 8, pl.ds((i % 8) * 16, 16)] = jnp.zeros(
                (16,), jnp.float32)
            return carry
        lax.fori_loop(0, _C * 8, _zb, 0)
        for k in range(rpt // 40):
            r0 = sid * rpt + k * 40
            pltpu.sync_copy(bufs[0].at[pl.ds(0, 40)], acc_sh.at[pl.ds(r0, 40)])
        idx_a.wait()
        idx_b.wait()
        plsc.subcore_barrier()

        for b in range(_NBUF - 1):
            pltpu.async_copy(h_hbm.at[sidx.at[b]], bufs[b], sems[b])

        def _grp(g, carry):
            j0 = g * _NBUF
            for b in range(_NBUF):
                j = j0 + b
                pf_buf = (b + _NBUF - 1) % _NBUF
                pltpu.async_copy(h_hbm.at[sidx.at[j + _NBUF - 1]],
                                 bufs[pf_buf], sems[pf_buf])
                pltpu.make_async_copy(h_hbm.at[sidx.at[j]],
                                      bufs[b], sems[b]).wait()
                pltpu.sync_copy(bufs[b], acc_sh.at[didx.at[j]], add=True)
            return carry
        lax.fori_loop(0, ngrp, _grp, 0)
        for t in range(rem):
            j = ngrp * _NBUF + t
            b = j % _NBUF
            pltpu.make_async_copy(h_hbm.at[sidx.at[j]],
                                  bufs[b], sems[b]).wait()
            pltpu.sync_copy(bufs[b], acc_sh.at[didx.at[j]], add=True)
        plsc.subcore_barrier()

        outs = []
        for k in range(nz):
            r0 = sid * rpt + k * _ZB
            outs.append(pltpu.async_copy(acc_sh.at[pl.ds(r0, _ZB)],
                                         acc_out.at[cid, pl.ds(r0, _ZB)],
                                         sem_a))
        for c in outs:
            c.wait()

    return pl.kernel(
        body,
        out_type=jax.ShapeDtypeStruct((_NC, _NPAD, 128), jnp.float32),
        mesh=mesh, scratch_types=scratch,
        compiler_params=pltpu.CompilerParams(use_tc_tiling_on_sc=False))


@functools.lru_cache(maxsize=None)
def _make_deg(per_w):
    rpt = _NPAD // _NS
    nz = rpt // _ZB

    scratch = [
        pltpu.VMEM_SHARED((_NPAD, 128), jnp.float32),
        pltpu.VMEM((per_w, _C), jnp.int32),
        pltpu.VMEM((_C, 128), jnp.float32),
    ]

    mesh = plsc.VectorSubcoreMesh(core_axis_name="c", subcore_axis_name="s")

    def body(dst_hbm, deg_out, deg_sh, didx, ones):
        cid = lax.axis_index("c")
        sid = lax.axis_index("s")
        wid = cid * _NS + sid

        def _fill(val):
            def _f(i, carry):
                ones[i // 8, pl.ds((i % 8) * 16, 16)] = jnp.full(
                    (16,), val, jnp.float32)
                return carry
            lax.fori_loop(0, _C * 8, _f, 0)

        _fill(0.0)
        for k in range(rpt // 40):
            r0 = sid * rpt + k * 40
            pltpu.sync_copy(ones.at[pl.ds(0, 40)], deg_sh.at[pl.ds(r0, 40)])
        _fill(1.0)
        plsc.subcore_barrier()

        pltpu.sync_copy(dst_hbm.at[wid], didx)

        def _step(j, carry):
            pltpu.sync_copy(ones, deg_sh.at[didx.at[j]], add=True)
            return carry
        lax.fori_loop(0, per_w, _step, 0)
        plsc.subcore_barrier()

        for k in range(nz):
            r0 = sid * rpt + k * _ZB
            pltpu.sync_copy(deg_sh.at[pl.ds(r0, _ZB)],
                            deg_out.at[cid, pl.ds(r0, _ZB)])

    return pl.kernel(
        body,
        out_type=jax.ShapeDtypeStruct((_NC, _NPAD, 128), jnp.float32),
        mesh=mesh, scratch_types=scratch)


_BLK = 1000


def _enc_call(x, w0, b0, w1, b1):
    n = x.shape[0]
    g = n // _BLK

    def body(x_ref, w0_ref, b0_ref, w1_ref, b1_ref, o_ref):
        h = jnp.maximum(
            jnp.dot(x_ref[...], w0_ref[...], preferred_element_type=jnp.float32)
            + b0_ref[...], 0.0)
        o_ref[...] = jnp.maximum(
            jnp.dot(h, w1_ref[...], preferred_element_type=jnp.float32)
            + b1_ref[...], 0.0)

    return pl.pallas_call(
        body,
        grid=(g,),
        in_specs=[
            pl.BlockSpec((_BLK, 128), lambda i: (i, 0)),
            pl.BlockSpec((128, 128), lambda i: (0, 0)),
            pl.BlockSpec((1, 128), lambda i: (0, 0)),
            pl.BlockSpec((128, 128), lambda i: (0, 0)),
            pl.BlockSpec((1, 128), lambda i: (0, 0)),
        ],
        out_specs=pl.BlockSpec((_BLK, 128), lambda i: (i, 0)),
        out_shape=jax.ShapeDtypeStruct((n, 128), jnp.float32),
    )(x, w0, b0, w1, b1)


def _conv1_call(acc3, deg3, h, w, b):
    n = h.shape[0]
    g = n // _BLK

    def body(m1, m2, d1, d2, h_ref, w_ref, b_ref, o_ref, inv_ref):
        inv = 1.0 / jnp.maximum(d1[0][:, :1] + d2[0][:, :1], 1.0)
        inv_ref[...] = jnp.broadcast_to(inv, (inv.shape[0], 8))
        m = (m1[0] + m2[0]) * inv
        o_ref[...] = jnp.maximum(
            jnp.dot(m, w_ref[...], preferred_element_type=jnp.float32)
            + b_ref[...] + h_ref[...], 0.0)

    return pl.pallas_call(
        body,
        grid=(g,),
        in_specs=[
            pl.BlockSpec((1, _BLK, 128), lambda i: (0, i, 0)),
            pl.BlockSpec((1, _BLK, 128), lambda i: (1, i, 0)),
            pl.BlockSpec((1, _BLK, 128), lambda i: (0, i, 0)),
            pl.BlockSpec((1, _BLK, 128), lambda i: (1, i, 0)),
            pl.BlockSpec((_BLK, 128), lambda i: (i, 0)),
            pl.BlockSpec((128, 128), lambda i: (0, 0)),
            pl.BlockSpec((1, 128), lambda i: (0, 0)),
        ],
        out_specs=[
            pl.BlockSpec((_BLK, 128), lambda i: (i, 0)),
            pl.BlockSpec((_BLK, 8), lambda i: (i, 0)),
        ],
        out_shape=[
            jax.ShapeDtypeStruct((n, 128), jnp.float32),
            jax.ShapeDtypeStruct((n, 8), jnp.float32),
        ],
    )(acc3, acc3, deg3, deg3, h, w, b)


def _conv_specs():
    return [
        pl.BlockSpec((1, _BLK, 128), lambda i: (0, i, 0)),
        pl.BlockSpec((1, _BLK, 128), lambda i: (1, i, 0)),
        pl.BlockSpec((_BLK, 8), lambda i: (i, 0)),
        pl.BlockSpec((_BLK, 128), lambda i: (i, 0)),
        pl.BlockSpec((128, 128), lambda i: (0, 0)),
        pl.BlockSpec((1, 128), lambda i: (0, 0)),
    ]


def _conv_call(acc3, invdeg, h, w, b):
    n = h.shape[0]
    g = n // _BLK

    def body(m1, m2, inv_ref, h_ref, w_ref, b_ref, o_ref):
        m = (m1[0] + m2[0]) * inv_ref[...][:, :1]
        o_ref[...] = jnp.maximum(
            jnp.dot(m, w_ref[...], preferred_element_type=jnp.float32)
            + b_ref[...] + h_ref[...], 0.0)

    return pl.pallas_call(
        body,
        grid=(g,),
        in_specs=_conv_specs(),
        out_specs=pl.BlockSpec((_BLK, 128), lambda i: (i, 0)),
        out_shape=jax.ShapeDtypeStruct((n, 128), jnp.float32),
    )(acc3, acc3, invdeg, h, w, b)


def _conv_pred_call(acc3, invdeg, h, w, b, pw0, pb0, pw1, pb1):
    n = h.shape[0]
    g = n // _BLK

    def body(m1, m2, inv_ref, h_ref, w_ref, b_ref,
             pw0_ref, pb0_ref, pw1_ref, pb1_ref, o_ref, acc_ref):
        i = pl.program_id(0)
        m = (m1[0] + m2[0]) * inv_ref[...][:, :1]
        h3 = jnp.maximum(
            jnp.dot(m, w_ref[...], preferred_element_type=jnp.float32)
            + b_ref[...] + h_ref[...], 0.0)

        @pl.when(i == 0)
        def _():
            acc_ref[...] = jnp.zeros_like(acc_ref)

        acc_ref[...] += jnp.sum(h3, axis=0, keepdims=True)

        @pl.when(i == g - 1)
        def _():
            obj = acc_ref[...] * (1.0 / n)
            z = jnp.maximum(
                jnp.dot(obj, pw0_ref[...], preferred_element_type=jnp.float32)
                + pb0_ref[...], 0.0)
            o_ref[...] = (jnp.dot(z, pw1_ref[...],
                                  preferred_element_type=jnp.float32)
                          + pb1_ref[...])

    specs = _conv_specs() + [
        pl.BlockSpec((128, 128), lambda i: (0, 0)),
        pl.BlockSpec((1, 128), lambda i: (0, 0)),
        pl.BlockSpec((128, 1), lambda i: (0, 0)),
        pl.BlockSpec((1, 1), lambda i: (0, 0)),
    ]
    return pl.pallas_call(
        body,
        grid=(g,),
        in_specs=specs,
        out_specs=pl.BlockSpec((1, 1), lambda i: (0, 0)),
        out_shape=jax.ShapeDtypeStruct((1, 1), jnp.float32),
        scratch_shapes=[pltpu.VMEM((1, 128), jnp.float32)],
    )(acc3, acc3, invdeg, h, w, b, pw0, pb0, pw1, pb1)


@jax.jit
def kernel(x, edge_index, enc_W0, enc_b0, enc_W1, enc_b1,
           conv_W0, conv_b0, conv_W1, conv_b1, conv_W2, conv_b2,
           pred_W0, pred_b0, pred_W1, pred_b1):
    e = edge_index.shape[1]
    per_w = e // (_C * _NW)
    src3 = edge_index[0].reshape(_NW, per_w, _C)
    dst3 = edge_index[1].reshape(_NW, per_w, _C)

    row = lambda v: v.reshape(1, -1)

    deg3 = _make_deg(per_w)(dst3)
    h = _enc_call(x, enc_W0, row(enc_b0), enc_W1, row(enc_b1))

    segsum = _make_segsum(per_w)

    acc3 = segsum(h, src3, dst3)
    h, invdeg = _conv1_call(acc3, deg3, h, conv_W0, row(conv_b0))
    acc3 = segsum(h, src3, dst3)
    h = _conv_call(acc3, invdeg, h, conv_W1, row(conv_b1))
    acc3 = segsum(h, src3, dst3)
    out = _conv_pred_call(acc3, invdeg, h, conv_W2, row(conv_b2),
                          pred_W0, row(pred_b0), pred_W1, row(pred_b1))
    return jnp.squeeze(out)


# speedup vs baseline: 1.0121x; 1.0121x over previous
import functools

import jax
import jax.numpy as jnp
from jax import lax
from jax.experimental import pallas as pl
from jax.experimental.pallas import tpu as pltpu
from jax.experimental.pallas import tpu_sc as plsc

_NC = 2
_NS = 16
_NW = _NC * _NS
_C = 100
_ZB = 128
_NPAD = 10240


@functools.lru_cache(maxsize=None)
def _make_segsum(per_w):
    rpt = _NPAD // _NS
    nz = rpt // _ZB

    scratch = [
        pltpu.VMEM_SHARED((_NPAD, 128), jnp.float32),
        pltpu.VMEM((per_w, _C), jnp.int32),
        pltpu.VMEM((per_w, _C), jnp.int32),
        pltpu.VMEM((_C, 128), jnp.float32),
        pltpu.VMEM((_C, 128), jnp.float32),
        pltpu.SemaphoreType.DMA,
        pltpu.SemaphoreType.DMA,
    ]

    mesh = plsc.VectorSubcoreMesh(core_axis_name="c", subcore_axis_name="s")

    half = per_w // 2
    assert per_w % 2 == 0

    def body(h_hbm, src_hbm, dst_hbm, acc_out,
             acc_sh, sidx, didx, buf_a, buf_b, sem_a, sem_b):
        cid = lax.axis_index("c")
        sid = lax.axis_index("s")
        wid = cid * _NS + sid

        idx_a = pltpu.async_copy(src_hbm.at[wid], sidx, sem_a)
        idx_b = pltpu.async_copy(dst_hbm.at[wid], didx, sem_b)

        def _zb(i, carry):
            buf_a[i // 8, pl.ds((i % 8) * 16, 16)] = jnp.zeros((16,), jnp.float32)
            return carry
        lax.fori_loop(0, _C * 8, _zb, 0)
        for k in range(rpt // 80):
            r0 = sid * rpt + k * 80
            pltpu.sync_copy(buf_a.at[pl.ds(0, 80)], acc_sh.at[pl.ds(r0, 80)])
        idx_a.wait()
        idx_b.wait()
        plsc.subcore_barrier()

        pltpu.async_copy(h_hbm.at[sidx.at[0]], buf_a, sem_a)

        def _step2(jj, carry):
            j0 = jj * 2
            j1 = j0 + 1
            pltpu.async_copy(h_hbm.at[sidx.at[j1]], buf_b, sem_b)
            pltpu.make_async_copy(h_hbm.at[sidx.at[j0]], buf_a, sem_a).wait()
            pltpu.sync_copy(buf_a, acc_sh.at[didx.at[j0]], add=True)

            @pl.when(jj < half - 1)
            def _():
                pltpu.async_copy(h_hbm.at[sidx.at[j0 + 2]], buf_a, sem_a)

            pltpu.make_async_copy(h_hbm.at[sidx.at[j1]], buf_b, sem_b).wait()
            pltpu.sync_copy(buf_b, acc_sh.at[didx.at[j1]], add=True)
            return carry
        lax.fori_loop(0, half, _step2, 0)
        plsc.subcore_barrier()

        outs = []
        for k in range(nz):
            r0 = sid * rpt + k * _ZB
            outs.append(pltpu.async_copy(acc_sh.at[pl.ds(r0, _ZB)],
                                         acc_out.at[cid, pl.ds(r0, _ZB)],
                                         sem_a))
        for c in outs:
            c.wait()

    return pl.kernel(
        body,
        out_type=jax.ShapeDtypeStruct((_NC, _NPAD, 128), jnp.float32),
        mesh=mesh, scratch_types=scratch,
        compiler_params=pltpu.CompilerParams(use_tc_tiling_on_sc=False))


@functools.lru_cache(maxsize=None)
def _make_deg(per_w):
    rpt = _NPAD // _NS
    nz = rpt // _ZB

    scratch = [
        pltpu.VMEM_SHARED((_NPAD, 128), jnp.float32),
        pltpu.VMEM((per_w, _C), jnp.int32),
        pltpu.VMEM((_C, 128), jnp.float32),
    ]

    mesh = plsc.VectorSubcoreMesh(core_axis_name="c", subcore_axis_name="s")

    def body(dst_hbm, deg_out, deg_sh, didx, ones):
        cid = lax.axis_index("c")
        sid = lax.axis_index("s")
        wid = cid * _NS + sid

        def _fill(val):
            def _f(i, carry):
                ones[i // 8, pl.ds((i % 8) * 16, 16)] = jnp.full(
                    (16,), val, jnp.float32)
                return carry
            lax.fori_loop(0, _C * 8, _f, 0)

        _fill(0.0)
        for k in range(rpt // 40):
            r0 = sid * rpt + k * 40
            pltpu.sync_copy(ones.at[pl.ds(0, 40)], deg_sh.at[pl.ds(r0, 40)])
        _fill(1.0)
        plsc.subcore_barrier()

        pltpu.sync_copy(dst_hbm.at[wid], didx)

        def _step(j, carry):
            pltpu.sync_copy(ones, deg_sh.at[didx.at[j]], add=True)
            return carry
        lax.fori_loop(0, per_w, _step, 0)
        plsc.subcore_barrier()

        for k in range(nz):
            r0 = sid * rpt + k * _ZB
            pltpu.sync_copy(deg_sh.at[pl.ds(r0, _ZB)],
                            deg_out.at[cid, pl.ds(r0, _ZB)])

    return pl.kernel(
        body,
        out_type=jax.ShapeDtypeStruct((_NC, _NPAD, 128), jnp.float32),
        mesh=mesh, scratch_types=scratch)


_BLK = 1000


def _enc_call(x, w0, b0, w1, b1):
    n = x.shape[0]
    g = n // _BLK

    def body(x_ref, w0_ref, b0_ref, w1_ref, b1_ref, o_ref):
        h = jnp.maximum(
            jnp.dot(x_ref[...], w0_ref[...], preferred_element_type=jnp.float32)
            + b0_ref[...], 0.0)
        o_ref[...] = jnp.maximum(
            jnp.dot(h, w1_ref[...], preferred_element_type=jnp.float32)
            + b1_ref[...], 0.0)

    return pl.pallas_call(
        body,
        grid=(g,),
        in_specs=[
            pl.BlockSpec((_BLK, 128), lambda i: (i, 0)),
            pl.BlockSpec((128, 128), lambda i: (0, 0)),
            pl.BlockSpec((1, 128), lambda i: (0, 0)),
            pl.BlockSpec((128, 128), lambda i: (0, 0)),
            pl.BlockSpec((1, 128), lambda i: (0, 0)),
        ],
        out_specs=pl.BlockSpec((_BLK, 128), lambda i: (i, 0)),
        out_shape=jax.ShapeDtypeStruct((n, 128), jnp.float32),
    )(x, w0, b0, w1, b1)


def _conv1_call(acc3, deg3, h, w, b):
    n = h.shape[0]
    g = n // _BLK

    def body(m1, m2, d1, d2, h_ref, w_ref, b_ref, o_ref, inv_ref):
        inv = 1.0 / jnp.maximum(d1[0][:, :1] + d2[0][:, :1], 1.0)
        inv_ref[...] = jnp.broadcast_to(inv, (inv.shape[0], 8))
        m = (m1[0] + m2[0]) * inv
        o_ref[...] = jnp.maximum(
            jnp.dot(m, w_ref[...], preferred_element_type=jnp.float32)
            + b_ref[...] + h_ref[...], 0.0)

    return pl.pallas_call(
        body,
        grid=(g,),
        in_specs=[
            pl.BlockSpec((1, _BLK, 128), lambda i: (0, i, 0)),
            pl.BlockSpec((1, _BLK, 128), lambda i: (1, i, 0)),
            pl.BlockSpec((1, _BLK, 128), lambda i: (0, i, 0)),
            pl.BlockSpec((1, _BLK, 128), lambda i: (1, i, 0)),
            pl.BlockSpec((_BLK, 128), lambda i: (i, 0)),
            pl.BlockSpec((128, 128), lambda i: (0, 0)),
            pl.BlockSpec((1, 128), lambda i: (0, 0)),
        ],
        out_specs=[
            pl.BlockSpec((_BLK, 128), lambda i: (i, 0)),
            pl.BlockSpec((_BLK, 8), lambda i: (i, 0)),
        ],
        out_shape=[
            jax.ShapeDtypeStruct((n, 128), jnp.float32),
            jax.ShapeDtypeStruct((n, 8), jnp.float32),
        ],
    )(acc3, acc3, deg3, deg3, h, w, b)


def _conv_specs():
    return [
        pl.BlockSpec((1, _BLK, 128), lambda i: (0, i, 0)),
        pl.BlockSpec((1, _BLK, 128), lambda i: (1, i, 0)),
        pl.BlockSpec((_BLK, 8), lambda i: (i, 0)),
        pl.BlockSpec((_BLK, 128), lambda i: (i, 0)),
        pl.BlockSpec((128, 128), lambda i: (0, 0)),
        pl.BlockSpec((1, 128), lambda i: (0, 0)),
    ]


def _conv_call(acc3, invdeg, h, w, b):
    n = h.shape[0]
    g = n // _BLK

    def body(m1, m2, inv_ref, h_ref, w_ref, b_ref, o_ref):
        m = (m1[0] + m2[0]) * inv_ref[...][:, :1]
        o_ref[...] = jnp.maximum(
            jnp.dot(m, w_ref[...], preferred_element_type=jnp.float32)
            + b_ref[...] + h_ref[...], 0.0)

    return pl.pallas_call(
        body,
        grid=(g,),
        in_specs=_conv_specs(),
        out_specs=pl.BlockSpec((_BLK, 128), lambda i: (i, 0)),
        out_shape=jax.ShapeDtypeStruct((n, 128), jnp.float32),
    )(acc3, acc3, invdeg, h, w, b)


def _conv_pred_call(acc3, invdeg, h, w, b, pw0, pb0, pw1, pb1):
    n = h.shape[0]
    g = n // _BLK

    def body(m1, m2, inv_ref, h_ref, w_ref, b_ref,
             pw0_ref, pb0_ref, pw1_ref, pb1_ref, o_ref, acc_ref):
        i = pl.program_id(0)
        m = (m1[0] + m2[0]) * inv_ref[...][:, :1]
        h3 = jnp.maximum(
            jnp.dot(m, w_ref[...], preferred_element_type=jnp.float32)
            + b_ref[...] + h_ref[...], 0.0)

        @pl.when(i == 0)
        def _():
            acc_ref[...] = jnp.zeros_like(acc_ref)

        acc_ref[...] += jnp.sum(h3, axis=0, keepdims=True)

        @pl.when(i == g - 1)
        def _():
            obj = acc_ref[...] * (1.0 / n)
            z = jnp.maximum(
                jnp.dot(obj, pw0_ref[...], preferred_element_type=jnp.float32)
                + pb0_ref[...], 0.0)
            o_ref[...] = (jnp.dot(z, pw1_ref[...],
                                  preferred_element_type=jnp.float32)
                          + pb1_ref[...])

    specs = _conv_specs() + [
        pl.BlockSpec((128, 128), lambda i: (0, 0)),
        pl.BlockSpec((1, 128), lambda i: (0, 0)),
        pl.BlockSpec((128, 1), lambda i: (0, 0)),
        pl.BlockSpec((1, 1), lambda i: (0, 0)),
    ]
    return pl.pallas_call(
        body,
        grid=(g,),
        in_specs=specs,
        out_specs=pl.BlockSpec((1, 1), lambda i: (0, 0)),
        out_shape=jax.ShapeDtypeStruct((1, 1), jnp.float32),
        scratch_shapes=[pltpu.VMEM((1, 128), jnp.float32)],
    )(acc3, acc3, invdeg, h, w, b, pw0, pb0, pw1, pb1)


@jax.jit
def kernel(x, edge_index, enc_W0, enc_b0, enc_W1, enc_b1,
           conv_W0, conv_b0, conv_W1, conv_b1, conv_W2, conv_b2,
           pred_W0, pred_b0, pred_W1, pred_b1):
    e = edge_index.shape[1]
    per_w = e // (_C * _NW)
    src3 = edge_index[0].reshape(_NW, per_w, _C)
    dst3 = edge_index[1].reshape(_NW, per_w, _C)

    row = lambda v: v.reshape(1, -1)

    deg3 = _make_deg(per_w)(dst3)
    h = _enc_call(x, enc_W0, row(enc_b0), enc_W1, row(enc_b1))

    segsum = _make_segsum(per_w)

    acc3 = segsum(h, src3, dst3)
    h, invdeg = _conv1_call(acc3, deg3, h, conv_W0, row(conv_b0))
    acc3 = segsum(h, src3, dst3)
    h = _conv_call(acc3, invdeg, h, conv_W1, row(conv_b1))
    acc3 = segsum(h, src3, dst3)
    out = _conv_pred_call(acc3, invdeg, h, conv_W2, row(conv_b2),
                          pred_W0, row(pred_b0), pred_W1, row(pred_b1))
    return jnp.squeeze(out)
